# baseline (speedup 1.0000x reference)
import jax
import jax.numpy as jnp
from jax import lax
from jax.experimental import pallas as pl
from jax.experimental.pallas import tpu as pltpu
from jax.experimental.pallas import tpu_sc as plsc

N = 10000
E = 160000
F = 256
H = 512
D = 256
C = 4
CW = H // C

NUM_CORES = 2
NUM_TILES = 16
EB = 128
NB_TILE = 79
EPAD = NUM_TILES * NB_TILE * EB
NBUF = 3
FULL_ROUNDS = (NB_TILE - (NBUF - 1)) // NBUF - 1
TAIL = NB_TILE - NBUF * (FULL_ROUNDS + 1)
PD = NBUF - 1
HALF = 5056
ACC_ROWS = 5120
ROWS_PER_TILE_Z = ACC_ROWS // NUM_TILES
OUT_TILE0 = 312
OUT_REM0 = HALF - NUM_TILES * OUT_TILE0
OUT_TILE1 = 304
OUT_REM1 = (N - HALF) - NUM_TILES * OUT_TILE1
OUT_PIECES0 = (104, 104, 104)
OUT_PIECES1 = (104, 104, 96)
STAGE_ROWS = 104


def _Z():
    return jnp.int32(0)

def _pre_body(x_ref, w1_ref, b1_ref, w2_ref, b2_ref, o_ref):
    xb = x_ref[...].astype(jnp.bfloat16)
    t = jnp.maximum(
        jnp.dot(xb, w1_ref[...], preferred_element_type=jnp.float32)
        + b1_ref[...], 0.0).astype(jnp.bfloat16)
    h = jnp.dot(t, w2_ref[...], preferred_element_type=jnp.float32) + b2_ref[...]
    for c in range(C):
        o_ref[c] = h[:, c * CW:(c + 1) * CW]


def _pre_call(x, w1, b1, w2, b2, rb=1000):
    grid = (N // rb,)
    return pl.pallas_call(
        _pre_body,
        grid=grid,
        in_specs=[
            pl.BlockSpec((rb, F), lambda i: (i, _Z())),
            pl.BlockSpec((F, H), lambda i: (_Z(), _Z())),
            pl.BlockSpec((1, H), lambda i: (_Z(), _Z())),
            pl.BlockSpec((H, H), lambda i: (_Z(), _Z())),
            pl.BlockSpec((1, H), lambda i: (_Z(), _Z())),
        ],
        out_specs=pl.BlockSpec((C, rb, CW), lambda i: (_Z(), i, _Z())),
        out_shape=jax.ShapeDtypeStruct((C, N, CW), jnp.float32),
    )(x, w1, b1, w2, b2)


def _cat(ref):
    return jnp.concatenate(
        [ref[c].astype(jnp.bfloat16) for c in range(C)], axis=-1)


def _conv_body(h_ref, a_ref, ws_ref, wn_ref, b_ref, o_ref):
    z = b_ref[...].astype(jnp.float32)
    z = z + jnp.dot(_cat(h_ref), ws_ref[...], preferred_element_type=jnp.float32)
    z = z + jnp.dot(_cat(a_ref), wn_ref[...], preferred_element_type=jnp.float32)
    z = jnp.maximum(z, 0.0)
    for c in range(C):
        o_ref[c] = z[:, c * CW:(c + 1) * CW]


def _conv_call(hc, ac, ws, wn, b, rb=1000):
    grid = (N // rb,)
    return pl.pallas_call(
        _conv_body,
        grid=grid,
        in_specs=[
            pl.BlockSpec((C, rb, CW), lambda i: (_Z(), i, _Z())),
            pl.BlockSpec((C, rb, CW), lambda i: (_Z(), i, _Z())),
            pl.BlockSpec((H, H), lambda i: (_Z(), _Z())),
            pl.BlockSpec((H, H), lambda i: (_Z(), _Z())),
            pl.BlockSpec((1, H), lambda i: (_Z(), _Z())),
        ],
        out_specs=pl.BlockSpec((C, rb, CW), lambda i: (_Z(), i, _Z())),
        out_shape=jax.ShapeDtypeStruct((C, N, CW), jnp.float32),
    )(hc, ac, ws, wn, b)


def _post_body(x_ref, h1_ref, h2_ref, h3_ref, h4_ref,
               wx_ref, w1_ref, w2_ref, w3_ref, w4_ref, b1_ref,
               wo_ref, bo_ref, o_ref):
    z = b1_ref[...].astype(jnp.float32)
    z = z + jnp.dot(x_ref[...].astype(jnp.bfloat16), wx_ref[...],
                    preferred_element_type=jnp.float32)
    for h_ref, w_ref in ((h1_ref, w1_ref), (h2_ref, w2_ref),
                         (h3_ref, w3_ref), (h4_ref, w4_ref)):
        z = z + jnp.dot(_cat(h_ref), w_ref[...],
                        preferred_element_type=jnp.float32)
    t = jnp.maximum(z, 0.0).astype(jnp.bfloat16)
    o_ref[...] = (jnp.dot(t, wo_ref[...], preferred_element_type=jnp.float32)
                  + bo_ref[...])


def _post_call(x, h1, h2, h3, h4, wx, w1, w2, w3, w4, b1, wo, bo, rb=400):
    grid = (N // rb,)
    hspec = pl.BlockSpec((C, rb, CW), lambda i: (_Z(), i, _Z()))
    wspec = pl.BlockSpec((H, H), lambda i: (_Z(), _Z()))
    return pl.pallas_call(
        _post_body,
        grid=grid,
        in_specs=[
            pl.BlockSpec((rb, F), lambda i: (i, _Z())),
            hspec, hspec, hspec, hspec,
            pl.BlockSpec((F, H), lambda i: (_Z(), _Z())),
            wspec, wspec, wspec, wspec,
            pl.BlockSpec((1, H), lambda i: (_Z(), _Z())),
            pl.BlockSpec((H, D), lambda i: (_Z(), _Z())),
            pl.BlockSpec((1, D), lambda i: (_Z(), _Z())),
        ],
        out_specs=pl.BlockSpec((rb, D), lambda i: (i, _Z())),
        out_shape=jax.ShapeDtypeStruct((N, D), jnp.float32),
    )(x, h1, h2, h3, h4, wx, w1, w2, w3, w4, b1, wo, bo)


def _seg_body(hc, srcr, dstr, zer, agg,
              idx_s, idx_d, rows, stage, acc, g0, g1, g2, g3):
    cid = lax.axis_index("c")
    sid = lax.axis_index("s")
    pltpu.sync_copy(srcr.at[sid], idx_s)
    pltpu.sync_copy(dstr.at[sid], idx_d)

    base = cid * jnp.int32(HALF)
    sz = jnp.where(cid == 0, jnp.int32(HALF), jnp.int32(N - HALF))

    def dst_local(r, carry):
        for cc in range(EB // 16):
            v = idx_d[r, pl.ds(cc * 16, 16)]
            l = v - base
            ok = (l >= jnp.int32(0)) & (l < sz)
            snk = sz + (v & jnp.int32(63))
            idx_d[r, pl.ds(cc * 16, 16)] = jnp.where(ok, l, snk)
        return carry

    lax.fori_loop(jnp.int32(0), jnp.int32(NB_TILE), dst_local, jnp.int32(0))

    for chunk in range(C):
        ci = jnp.int32(chunk)
        pltpu.sync_copy(zer.at[pl.ds(sid * ROWS_PER_TILE_Z, ROWS_PER_TILE_Z)],
                        acc.at[pl.ds(sid * ROWS_PER_TILE_Z, ROWS_PER_TILE_Z)])
        plsc.subcore_barrier()

        sems = (g0, g1, g2, g3)[:NBUF]

        def start_gather(j, b):
            pltpu.async_copy(hc.at[ci].at[idx_s.at[j]], rows.at[jnp.int32(b)],
                             sems[b])

        def wait_gather(b):
            pltpu.make_async_copy(hc.at[jnp.int32(0), pl.ds(0, EB)],
                                  rows.at[jnp.int32(b)], sems[b]).wait()

        def do_scatter(j, b):
            pltpu.sync_copy(rows.at[jnp.int32(b)], acc.at[idx_d.at[j]],
                            add=True)

        for b in range(PD):
            start_gather(jnp.int32(b), b)

        for b in range(NBUF):
            j = jnp.int32(b)
            wait_gather(b)
            start_gather(j + PD, (b + PD) % NBUF)
            do_scatter(j, b)

        def round_body(g, carry):
            for b in range(NBUF):
                j = g * NBUF + b
                wait_gather(b)
                start_gather(j + PD, (b + PD) % NBUF)
                do_scatter(j, b)
            return carry

        lax.fori_loop(jnp.int32(1), jnp.int32(FULL_ROUNDS + 1),
                      round_body, jnp.int32(0))

        for t in range(TAIL):
            j = NBUF * (FULL_ROUNDS + 1) + t
            b = j % NBUF
            wait_gather(b)
            if j + PD < NB_TILE:
                start_gather(jnp.int32(j + PD), (b + PD) % NBUF)
            do_scatter(jnp.int32(j), b)
        plsc.subcore_barrier()

        @pl.when(cid == 0)
        def _():
            base = sid * OUT_TILE0
            off = 0
            for npiece in OUT_PIECES0:
                st = stage.at[pl.ds(0, npiece)]
                pltpu.sync_copy(acc.at[pl.ds(base + off, npiece)], st)
                pltpu.sync_copy(st, agg.at[ci, pl.ds(base + off, npiece)])
                off += npiece

        @pl.when((cid == 0) & (sid == 0))
        def _():
            r0 = NUM_TILES * OUT_TILE0
            st = stage.at[pl.ds(0, OUT_REM0)]
            pltpu.sync_copy(acc.at[pl.ds(r0, OUT_REM0)], st)
            pltpu.sync_copy(st, agg.at[ci, pl.ds(r0, OUT_REM0)])

        @pl.when(cid == 1)
        def _():
            base = sid * OUT_TILE1
            off = 0
            for npiece in OUT_PIECES1:
                st = stage.at[pl.ds(0, npiece)]
                pltpu.sync_copy(acc.at[pl.ds(base + off, npiece)], st)
                pltpu.sync_copy(st, agg.at[ci, pl.ds(HALF + base + off, npiece)])
                off += npiece

        @pl.when((cid == 1) & (sid == 0))
        def _():
            r0 = NUM_TILES * OUT_TILE1
            st = stage.at[pl.ds(0, OUT_REM1)]
            pltpu.sync_copy(acc.at[pl.ds(r0, OUT_REM1)], st)
            pltpu.sync_copy(st, agg.at[ci, pl.ds(HALF + r0, OUT_REM1)])

        plsc.subcore_barrier()


def _seg_call(hc, srcr, dstr, zer):
    mesh = plsc.VectorSubcoreMesh(core_axis_name="c", subcore_axis_name="s")
    f = pl.kernel(
        _seg_body,
        out_type=jax.ShapeDtypeStruct((C, N, CW), jnp.float32),
        mesh=mesh,
        scratch_types=[
            pltpu.VMEM((NB_TILE, EB), jnp.int32),
            pltpu.VMEM((NB_TILE, EB), jnp.int32),
            pltpu.VMEM((NBUF, EB, CW), jnp.float32),
            pltpu.VMEM((STAGE_ROWS, CW), jnp.float32),
            pltpu.VMEM_SHARED((ACC_ROWS, CW), jnp.float32),
        ] + [pltpu.SemaphoreType.DMA] * 4,
    )
    return f(hc, srcr, dstr, zer)


def kernel(x, edge_index, pre_w1, pre_b1, pre_w2, pre_b2,
           conv0_ws, conv0_wn, conv0_b, conv1_ws, conv1_wn, conv1_b,
           conv2_ws, conv2_wn, conv2_b, post_w1, post_b1, post_w2, post_b2):
    src = edge_index[0].astype(jnp.int32)
    dst = edge_index[1].astype(jnp.int32)
    pad = EPAD - E
    src_p = jnp.concatenate([src, jnp.zeros((pad,), jnp.int32)])
    dst_p = jnp.concatenate([dst, jnp.full((pad,), N, jnp.int32)])
    srcr = src_p.reshape(NUM_TILES, NB_TILE, EB)
    dstr = dst_p.reshape(NUM_TILES, NB_TILE, EB)
    zer = jnp.zeros((ACC_ROWS, CW), jnp.float32)

    bf = jnp.bfloat16
    b_pre1 = pre_b1.reshape(1, H)
    b_pre2 = pre_b2.reshape(1, H)

    h1 = _pre_call(x, pre_w1.astype(bf), b_pre1, pre_w2.astype(bf), b_pre2)

    hs = [h1]
    h = h1
    for ws, wn, b in ((conv0_ws, conv0_wn, conv0_b),
                      (conv1_ws, conv1_wn, conv1_b),
                      (conv2_ws, conv2_wn, conv2_b)):
        agg = _seg_call(h, srcr, dstr, zer)
        h = _conv_call(h, agg, ws.astype(bf), wn.astype(bf), b.reshape(1, H))
        hs.append(h)

    wx = post_w1[:F].astype(bf)
    wh = [post_w1[F + l * H:F + (l + 1) * H].astype(bf) for l in range(4)]
    out = _post_call(x, hs[0], hs[1], hs[2], hs[3],
                     wx, wh[0], wh[1], wh[2], wh[3],
                     post_b1.reshape(1, H), post_w2.astype(bf),
                     post_b2.reshape(1, D))
    return out


# speedup vs baseline: 1.2515x; 1.2515x over previous
import jax
import jax.numpy as jnp
from jax import lax
from jax.experimental import pallas as pl
from jax.experimental.pallas import tpu as pltpu
from jax.experimental.pallas import tpu_sc as plsc

N = 10000
E = 160000
F = 256
H = 512
D = 256
C = 4
CW = H // C

NUM_CORES = 2
NUM_TILES = 16
EB = 128
NB_TILE = 79
EPAD = NUM_TILES * NB_TILE * EB
NBUF = 3
FULL_ROUNDS = (NB_TILE - (NBUF - 1)) // NBUF - 1
TAIL = NB_TILE - NBUF * (FULL_ROUNDS + 1)
PD = NBUF - 1
HALF = 5056
ACC_ROWS = 5120
ROWS_PER_TILE_Z = ACC_ROWS // NUM_TILES
OUT_TILE0 = 312
OUT_REM0 = HALF - NUM_TILES * OUT_TILE0
OUT_TILE1 = 304
OUT_REM1 = (N - HALF) - NUM_TILES * OUT_TILE1
OUT_PIECES0 = (104, 104, 104)
OUT_PIECES1 = (104, 104, 96)
STAGE_ROWS = 104


def _Z():
    return jnp.int32(0)

def _pre_body(x_ref, w1_ref, b1_ref, w2_ref, b2_ref, o_ref):
    xb = x_ref[...].astype(jnp.bfloat16)
    t = jnp.maximum(
        jnp.dot(xb, w1_ref[...], preferred_element_type=jnp.float32)
        + b1_ref[...], 0.0).astype(jnp.bfloat16)
    h = jnp.dot(t, w2_ref[...], preferred_element_type=jnp.float32) + b2_ref[...]
    for c in range(C):
        o_ref[c] = h[:, c * CW:(c + 1) * CW]


def _pre_call(x, w1, b1, w2, b2, rb=1000):
    grid = (N // rb,)
    return pl.pallas_call(
        _pre_body,
        grid=grid,
        in_specs=[
            pl.BlockSpec((rb, F), lambda i: (i, _Z())),
            pl.BlockSpec((F, H), lambda i: (_Z(), _Z())),
            pl.BlockSpec((1, H), lambda i: (_Z(), _Z())),
            pl.BlockSpec((H, H), lambda i: (_Z(), _Z())),
            pl.BlockSpec((1, H), lambda i: (_Z(), _Z())),
        ],
        out_specs=pl.BlockSpec((C, rb, CW), lambda i: (_Z(), i, _Z())),
        out_shape=jax.ShapeDtypeStruct((C, N, CW), jnp.float32),
    )(x, w1, b1, w2, b2)


def _cat(ref):
    return jnp.concatenate(
        [ref[c].astype(jnp.bfloat16) for c in range(C)], axis=-1)


def _conv_body(h_ref, a_ref, ws_ref, wn_ref, b_ref, o_ref):
    z = b_ref[...].astype(jnp.float32)
    z = z + jnp.dot(_cat(h_ref), ws_ref[...], preferred_element_type=jnp.float32)
    z = z + jnp.dot(_cat(a_ref), wn_ref[...], preferred_element_type=jnp.float32)
    z = jnp.maximum(z, 0.0)
    for c in range(C):
        o_ref[c] = z[:, c * CW:(c + 1) * CW]


def _conv_call(hc, ac, ws, wn, b, rb=1000):
    grid = (N // rb,)
    return pl.pallas_call(
        _conv_body,
        grid=grid,
        in_specs=[
            pl.BlockSpec((C, rb, CW), lambda i: (_Z(), i, _Z())),
            pl.BlockSpec((C, rb, CW), lambda i: (_Z(), i, _Z())),
            pl.BlockSpec((H, H), lambda i: (_Z(), _Z())),
            pl.BlockSpec((H, H), lambda i: (_Z(), _Z())),
            pl.BlockSpec((1, H), lambda i: (_Z(), _Z())),
        ],
        out_specs=pl.BlockSpec((C, rb, CW), lambda i: (_Z(), i, _Z())),
        out_shape=jax.ShapeDtypeStruct((C, N, CW), jnp.float32),
    )(hc, ac, ws, wn, b)


def _post_body(x_ref, h1_ref, h2_ref, h3_ref, h4_ref,
               wx_ref, w1_ref, w2_ref, w3_ref, w4_ref, b1_ref,
               wo_ref, bo_ref, o_ref):
    z = b1_ref[...].astype(jnp.float32)
    z = z + jnp.dot(x_ref[...].astype(jnp.bfloat16), wx_ref[...],
                    preferred_element_type=jnp.float32)
    for h_ref, w_ref in ((h1_ref, w1_ref), (h2_ref, w2_ref),
                         (h3_ref, w3_ref), (h4_ref, w4_ref)):
        z = z + jnp.dot(_cat(h_ref), w_ref[...],
                        preferred_element_type=jnp.float32)
    t = jnp.maximum(z, 0.0).astype(jnp.bfloat16)
    o_ref[...] = (jnp.dot(t, wo_ref[...], preferred_element_type=jnp.float32)
                  + bo_ref[...])


def _post_call(x, h1, h2, h3, h4, wx, w1, w2, w3, w4, b1, wo, bo, rb=400):
    grid = (N // rb,)
    hspec = pl.BlockSpec((C, rb, CW), lambda i: (_Z(), i, _Z()))
    wspec = pl.BlockSpec((H, H), lambda i: (_Z(), _Z()))
    return pl.pallas_call(
        _post_body,
        grid=grid,
        in_specs=[
            pl.BlockSpec((rb, F), lambda i: (i, _Z())),
            hspec, hspec, hspec, hspec,
            pl.BlockSpec((F, H), lambda i: (_Z(), _Z())),
            wspec, wspec, wspec, wspec,
            pl.BlockSpec((1, H), lambda i: (_Z(), _Z())),
            pl.BlockSpec((H, D), lambda i: (_Z(), _Z())),
            pl.BlockSpec((1, D), lambda i: (_Z(), _Z())),
        ],
        out_specs=pl.BlockSpec((rb, D), lambda i: (i, _Z())),
        out_shape=jax.ShapeDtypeStruct((N, D), jnp.float32),
    )(x, h1, h2, h3, h4, wx, w1, w2, w3, w4, b1, wo, bo)


def _seg_body(hc, srcr, dstr, zer, agg,
              idx_s, idx_d, rows, stage, acc, g0, g1, g2, g3):
    cid = lax.axis_index("c")
    sid = lax.axis_index("s")
    pltpu.sync_copy(srcr.at[sid], idx_s)
    pltpu.sync_copy(dstr.at[sid], idx_d)

    base = cid * jnp.int32(HALF)
    sz = jnp.where(cid == 0, jnp.int32(HALF), jnp.int32(N - HALF))

    def dst_local(r, carry):
        for cc in range(EB // 16):
            v = idx_d[r, pl.ds(cc * 16, 16)]
            l = v - base
            ok = (l >= jnp.int32(0)) & (l < sz)
            snk = sz + (v & jnp.int32(63))
            idx_d[r, pl.ds(cc * 16, 16)] = jnp.where(ok, l, snk)
        return carry

    lax.fori_loop(jnp.int32(0), jnp.int32(NB_TILE), dst_local, jnp.int32(0))

    for chunk in range(C):
        ci = (jnp.int32(chunk) + cid * jnp.int32(2)) % jnp.int32(C)
        pltpu.sync_copy(zer.at[pl.ds(sid * ROWS_PER_TILE_Z, ROWS_PER_TILE_Z)],
                        acc.at[pl.ds(sid * ROWS_PER_TILE_Z, ROWS_PER_TILE_Z)])
        plsc.subcore_barrier()

        sems = (g0, g1, g2, g3)[:NBUF]

        def start_gather(j, b):
            pltpu.async_copy(hc.at[ci].at[idx_s.at[j]], rows.at[jnp.int32(b)],
                             sems[b])

        def wait_gather(b):
            pltpu.make_async_copy(hc.at[jnp.int32(0), pl.ds(0, EB)],
                                  rows.at[jnp.int32(b)], sems[b]).wait()

        def do_scatter(j, b):
            pltpu.sync_copy(rows.at[jnp.int32(b)], acc.at[idx_d.at[j]],
                            add=True)

        for b in range(PD):
            start_gather(jnp.int32(b), b)

        for b in range(NBUF):
            j = jnp.int32(b)
            wait_gather(b)
            start_gather(j + PD, (b + PD) % NBUF)
            do_scatter(j, b)

        def round_body(g, carry):
            for b in range(NBUF):
                j = g * NBUF + b
                wait_gather(b)
                start_gather(j + PD, (b + PD) % NBUF)
                do_scatter(j, b)
            return carry

        lax.fori_loop(jnp.int32(1), jnp.int32(FULL_ROUNDS + 1),
                      round_body, jnp.int32(0))

        for t in range(TAIL):
            j = NBUF * (FULL_ROUNDS + 1) + t
            b = j % NBUF
            wait_gather(b)
            if j + PD < NB_TILE:
                start_gather(jnp.int32(j + PD), (b + PD) % NBUF)
            do_scatter(jnp.int32(j), b)
        plsc.subcore_barrier()

        @pl.when(cid == 0)
        def _():
            base = sid * OUT_TILE0
            off = 0
            for npiece in OUT_PIECES0:
                st = stage.at[pl.ds(0, npiece)]
                pltpu.sync_copy(acc.at[pl.ds(base + off, npiece)], st)
                pltpu.sync_copy(st, agg.at[ci, pl.ds(base + off, npiece)])
                off += npiece

        @pl.when((cid == 0) & (sid == 0))
        def _():
            r0 = NUM_TILES * OUT_TILE0
            st = stage.at[pl.ds(0, OUT_REM0)]
            pltpu.sync_copy(acc.at[pl.ds(r0, OUT_REM0)], st)
            pltpu.sync_copy(st, agg.at[ci, pl.ds(r0, OUT_REM0)])

        @pl.when(cid == 1)
        def _():
            base = sid * OUT_TILE1
            off = 0
            for npiece in OUT_PIECES1:
                st = stage.at[pl.ds(0, npiece)]
                pltpu.sync_copy(acc.at[pl.ds(base + off, npiece)], st)
                pltpu.sync_copy(st, agg.at[ci, pl.ds(HALF + base + off, npiece)])
                off += npiece

        @pl.when((cid == 1) & (sid == 0))
        def _():
            r0 = NUM_TILES * OUT_TILE1
            st = stage.at[pl.ds(0, OUT_REM1)]
            pltpu.sync_copy(acc.at[pl.ds(r0, OUT_REM1)], st)
            pltpu.sync_copy(st, agg.at[ci, pl.ds(HALF + r0, OUT_REM1)])

        plsc.subcore_barrier()


def _seg_call(hc, srcr, dstr, zer):
    mesh = plsc.VectorSubcoreMesh(core_axis_name="c", subcore_axis_name="s")
    f = pl.kernel(
        _seg_body,
        out_type=jax.ShapeDtypeStruct((C, N, CW), jnp.float32),
        mesh=mesh,
        scratch_types=[
            pltpu.VMEM((NB_TILE, EB), jnp.int32),
            pltpu.VMEM((NB_TILE, EB), jnp.int32),
            pltpu.VMEM((NBUF, EB, CW), jnp.float32),
            pltpu.VMEM((STAGE_ROWS, CW), jnp.float32),
            pltpu.VMEM_SHARED((ACC_ROWS, CW), jnp.float32),
        ] + [pltpu.SemaphoreType.DMA] * 4,
    )
    return f(hc, srcr, dstr, zer)


def kernel(x, edge_index, pre_w1, pre_b1, pre_w2, pre_b2,
           conv0_ws, conv0_wn, conv0_b, conv1_ws, conv1_wn, conv1_b,
           conv2_ws, conv2_wn, conv2_b, post_w1, post_b1, post_w2, post_b2):
    src = edge_index[0].astype(jnp.int32)
    dst = edge_index[1].astype(jnp.int32)
    pad = EPAD - E
    src_p = jnp.concatenate([src, jnp.zeros((pad,), jnp.int32)])
    dst_p = jnp.concatenate([dst, jnp.full((pad,), N, jnp.int32)])
    srcr = src_p.reshape(NUM_TILES, NB_TILE, EB)
    dstr = dst_p.reshape(NUM_TILES, NB_TILE, EB)
    zer = jnp.zeros((ACC_ROWS, CW), jnp.float32)

    bf = jnp.bfloat16
    b_pre1 = pre_b1.reshape(1, H)
    b_pre2 = pre_b2.reshape(1, H)

    h1 = _pre_call(x, pre_w1.astype(bf), b_pre1, pre_w2.astype(bf), b_pre2)

    hs = [h1]
    h = h1
    for ws, wn, b in ((conv0_ws, conv0_wn, conv0_b),
                      (conv1_ws, conv1_wn, conv1_b),
                      (conv2_ws, conv2_wn, conv2_b)):
        agg = _seg_call(h, srcr, dstr, zer)
        h = _conv_call(h, agg, ws.astype(bf), wn.astype(bf), b.reshape(1, H))
        hs.append(h)

    wx = post_w1[:F].astype(bf)
    wh = [post_w1[F + l * H:F + (l + 1) * H].astype(bf) for l in range(4)]
    out = _post_call(x, hs[0], hs[1], hs[2], hs[3],
                     wx, wh[0], wh[1], wh[2], wh[3],
                     post_b1.reshape(1, H), post_w2.astype(bf),
                     post_b2.reshape(1, D))
    return out
